# baseline (speedup 1.0000x reference)
import jax
import jax.numpy as jnp
from jax.experimental import pallas as pl
from jax.experimental.pallas import tpu as pltpu
from jax.experimental.pallas import tpu_sc as plsc

VOCAB = 100000
EMBED = 128
BATCH = 1024

GATHER_WINDOW = 128
VTILE = 5000


def _sc_gather(table, x):
    mesh = plsc.VectorSubcoreMesh(core_axis_name="c", subcore_axis_name="s")
    indices = x.reshape(2, BATCH // 2)
    steps_per_core = BATCH // GATHER_WINDOW // 2

    @pl.kernel(
        out_type=jax.ShapeDtypeStruct((BATCH, EMBED), table.dtype),
        mesh=mesh,
    )
    def gather_kernel(table_hbm, idx_hbm, out_hbm):
        def body(idx_vmem, out_vmem):
            pltpu.sync_copy(table_hbm.at[idx_vmem.at[0]], out_vmem)

        pltpu.emit_pipeline(
            body,
            grid=(2, steps_per_core),
            in_specs=[pl.BlockSpec((1, GATHER_WINDOW),
                                   index_map=lambda c, i: (c, i))],
            out_specs=[pl.BlockSpec((GATHER_WINDOW, EMBED),
                                    index_map=lambda c, i: (c * steps_per_core + i, 0))],
            core_axis_name=("c", "s"),
            dimension_semantics=(pltpu.PARALLEL, pltpu.PARALLEL),
        )(idx_hbm, out_hbm)

    return gather_kernel(table, indices)


def _matmul_body(w_ref, e_ref, b_ref, o_ref):
    bias_col = b_ref[0].T
    o_ref[...] = jax.lax.dot_general(
        w_ref[...], e_ref[...],
        dimension_numbers=(((1,), (1,)), ((), ())),
        preferred_element_type=jnp.float32,
    ) + bias_col


def _tc_matmul(embed, W, b):
    outT = pl.pallas_call(
        _matmul_body,
        grid=(VOCAB // VTILE,),
        in_specs=[
            pl.BlockSpec((VTILE, EMBED), lambda j: (j, 0)),
            pl.BlockSpec((BATCH, EMBED), lambda j: (0, 0)),
            pl.BlockSpec((1, 1, VTILE), lambda j: (j, 0, 0)),
        ],
        out_specs=pl.BlockSpec((VTILE, BATCH), lambda j: (j, 0)),
        out_shape=jax.ShapeDtypeStruct((VOCAB, BATCH), jnp.float32),
        compiler_params=pltpu.CompilerParams(
            dimension_semantics=("parallel",),
            vmem_limit_bytes=100 * 1024 * 1024,
        ),
    )(W, embed, b.reshape(VOCAB // VTILE, 1, VTILE))
    return outT.T


def kernel(x, table, W, b):
    embed = _sc_gather(table, x)
    return _tc_matmul(embed, W, b)


# speedup vs baseline: 1.0119x; 1.0119x over previous
import jax
import jax.numpy as jnp
from jax.experimental import pallas as pl
from jax.experimental.pallas import tpu as pltpu
from jax.experimental.pallas import tpu_sc as plsc

VOCAB = 100000
EMBED = 128
BATCH = 1024

GATHER_WINDOW = 128
VTILE = 6144


def _sc_gather(table, x):
    mesh = plsc.VectorSubcoreMesh(core_axis_name="c", subcore_axis_name="s")
    indices = x.reshape(1, BATCH)

    @pl.kernel(
        out_type=jax.ShapeDtypeStruct((BATCH, EMBED), table.dtype),
        mesh=mesh,
    )
    def gather_kernel(table_hbm, idx_hbm, out_hbm):
        def body(idx_vmem, out_vmem):
            pltpu.sync_copy(table_hbm.at[idx_vmem.at[0]], out_vmem)

        pltpu.emit_pipeline(
            body,
            grid=(BATCH // GATHER_WINDOW,),
            in_specs=[pl.BlockSpec((1, GATHER_WINDOW), index_map=lambda i: (0, i))],
            out_specs=[pl.BlockSpec((GATHER_WINDOW, EMBED), index_map=lambda i: (i, 0))],
            core_axis_name=("c", "s"),
            dimension_semantics=(pltpu.PARALLEL,),
        )(idx_hbm, out_hbm)

    return gather_kernel(table, indices)


def _bias_blocks(b):
    nblk = pl.cdiv(VOCAB, VTILE)
    bpad = jnp.pad(b, (0, nblk * VTILE - VOCAB))
    return bpad.reshape(nblk, 1, VTILE)


def _matmul_body(w_ref, e_ref, b_ref, o_ref):
    bias_col = b_ref[0].T
    o_ref[...] = jax.lax.dot_general(
        w_ref[...], e_ref[...],
        dimension_numbers=(((1,), (1,)), ((), ())),
        preferred_element_type=jnp.float32,
    ) + bias_col


def _tc_matmul(embed, W, b):
    outT = pl.pallas_call(
        _matmul_body,
        grid=(pl.cdiv(VOCAB, VTILE),),
        in_specs=[
            pl.BlockSpec((VTILE, EMBED), lambda j: (j, 0)),
            pl.BlockSpec((BATCH, EMBED), lambda j: (0, 0)),
            pl.BlockSpec((1, 1, VTILE), lambda j: (j, 0, 0)),
        ],
        out_specs=pl.BlockSpec((VTILE, BATCH), lambda j: (j, 0)),
        out_shape=jax.ShapeDtypeStruct((VOCAB, BATCH), jnp.float32),
        compiler_params=pltpu.CompilerParams(
            dimension_semantics=("parallel",),
            vmem_limit_bytes=100 * 1024 * 1024,
        ),
    )(W, embed, _bias_blocks(b))
    return outT.T


def kernel(x, table, W, b):
    embed = _sc_gather(table, x)
    return _tc_matmul(embed, W, b)


# speedup vs baseline: 1.0232x; 1.0111x over previous
import functools
import jax
import jax.numpy as jnp
from jax.experimental import pallas as pl
from jax.experimental.pallas import tpu as pltpu
from jax.experimental.pallas import tpu_sc as plsc

VOCAB = 100000
EMBED = 128
BATCH = 1024

VTILE = 6144


def _sc_gather(table, x):
    mesh = plsc.VectorSubcoreMesh(core_axis_name="c", subcore_axis_name="s")
    num_cores, num_subcores = 2, 16
    b_per_w = BATCH // (num_cores * num_subcores)

    @functools.partial(
        pl.kernel,
        out_type=jax.ShapeDtypeStruct((BATCH, EMBED), table.dtype),
        mesh=mesh,
        scratch_types=[
            pltpu.VMEM((b_per_w,), jnp.int32),
            pltpu.VMEM((b_per_w, EMBED), jnp.float32),
            pltpu.SemaphoreType.DMA,
        ],
    )
    def gather_kernel(table_hbm, idx_hbm, out_hbm, idx_v, rows_v, sem):
        wid = jax.lax.axis_index("s") * num_cores + jax.lax.axis_index("c")
        base = wid * b_per_w
        pltpu.sync_copy(idx_hbm.at[pl.ds(base, b_per_w)], idx_v)
        pltpu.async_copy(table_hbm.at[idx_v], rows_v, sem).wait()
        pltpu.sync_copy(rows_v, out_hbm.at[pl.ds(base, b_per_w)])

    return gather_kernel(table, x)


def _bias_blocks(b):
    nblk = pl.cdiv(VOCAB, VTILE)
    bpad = jnp.pad(b, (0, nblk * VTILE - VOCAB))
    return bpad.reshape(nblk, 1, VTILE)


def _matmul_body(w_ref, e_ref, b_ref, o_ref):
    bias_col = b_ref[0].T
    o_ref[...] = jax.lax.dot_general(
        w_ref[...], e_ref[...],
        dimension_numbers=(((1,), (1,)), ((), ())),
        preferred_element_type=jnp.float32,
    ) + bias_col


def _tc_matmul(embed, W, b):
    outT = pl.pallas_call(
        _matmul_body,
        grid=(pl.cdiv(VOCAB, VTILE),),
        in_specs=[
            pl.BlockSpec((VTILE, EMBED), lambda j: (j, 0)),
            pl.BlockSpec((BATCH, EMBED), lambda j: (0, 0)),
            pl.BlockSpec((1, 1, VTILE), lambda j: (j, 0, 0)),
        ],
        out_specs=pl.BlockSpec((VTILE, BATCH), lambda j: (j, 0)),
        out_shape=jax.ShapeDtypeStruct((VOCAB, BATCH), jnp.float32),
        compiler_params=pltpu.CompilerParams(
            dimension_semantics=("parallel",),
            vmem_limit_bytes=100 * 1024 * 1024,
        ),
    )(W, embed, _bias_blocks(b))
    return outT.T


def kernel(x, table, W, b):
    embed = _sc_gather(table, x)
    return _tc_matmul(embed, W, b)
